# baseline (speedup 1.0000x reference)
import jax
import jax.numpy as jnp
from jax import lax
from jax.experimental import pallas as pl
from jax.experimental.pallas import tpu as pltpu

ROWS = 128
COLS = 32768
BW = 16384
GRID = COLS // BW


def _tc_body(x_ref, o_ref):
    v = x_ref[...]
    ridx = lax.broadcasted_iota(jnp.int32, (ROWS, BW), 0)
    mx = jnp.max(v, axis=0, keepdims=True)
    cand = jnp.where(v == mx, ridx, jnp.int32(ROWS))
    o_ref[...] = jnp.min(cand, axis=0, keepdims=True)


@jax.jit
def _argmax_tc(x):
    return pl.pallas_call(
        _tc_body,
        grid=(GRID,),
        in_specs=[pl.BlockSpec((ROWS, BW), lambda i: (0, i))],
        out_specs=pl.BlockSpec((1, BW), lambda i: (0, i)),
        out_shape=jax.ShapeDtypeStruct((1, COLS), jnp.int32),
    )(x)


def kernel(x):
    return _argmax_tc(x).astype(jnp.int64)


# speedup vs baseline: 1.1597x; 1.1597x over previous
import jax
import jax.numpy as jnp
from jax import lax
from jax.experimental import pallas as pl
from jax.experimental.pallas import tpu as pltpu

ROWS = 128
COLS = 32768
CW = 4096
NCH = COLS // CW
NBUF = 4


def _tc_body(x_hbm, o_ref, *rest):
    bufs = rest[:NBUF]
    sems = rest[NBUF:]

    def dma(i):
        return pltpu.make_async_copy(
            x_hbm.at[:, pl.ds(i * CW, CW)], bufs[i % NBUF], sems[i % NBUF])

    for i in range(min(NBUF, NCH)):
        dma(i).start()
    for i in range(NCH):
        dma(i).wait()
        v = bufs[i % NBUF][...]
        ridx = lax.broadcasted_iota(jnp.int32, (ROWS, CW), 0)
        mx = jnp.max(v, axis=0, keepdims=True)
        cand = jnp.where(v == mx, ridx, jnp.int32(ROWS))
        o_ref[:, pl.ds(i * CW, CW)] = jnp.min(cand, axis=0, keepdims=True)
        if i + NBUF < NCH:
            dma(i + NBUF).start()


@jax.jit
def _argmax_tc(x):
    return pl.pallas_call(
        _tc_body,
        in_specs=[pl.BlockSpec(memory_space=pltpu.MemorySpace.HBM)],
        out_specs=pl.BlockSpec(memory_space=pltpu.MemorySpace.VMEM),
        out_shape=jax.ShapeDtypeStruct((1, COLS), jnp.int32),
        scratch_shapes=(
            [pltpu.VMEM((ROWS, CW), jnp.float32) for _ in range(NBUF)]
            + [pltpu.SemaphoreType.DMA for _ in range(NBUF)]
        ),
    )(x)


def kernel(x):
    return _argmax_tc(x).astype(jnp.int64)
